# Initial kernel scaffold; baseline (speedup 1.0000x reference)
import jax
import jax.numpy as jnp
from jax.experimental import pallas as pl


def kernel(input, init_hidden, W_q, W_k, W_v, W_x, W_h, b_x, b_h):
    raise NotImplementedError("write your pallas kernel here")


# speedup vs baseline: 1.8484x; 1.8484x over previous
import functools
import math

import jax
import jax.numpy as jnp
from jax.experimental import pallas as pl

B = 128
S = 32
D_IN = 512
HIDDEN = 512
D_K = 512
D_V = 512
NUM_K = 16
ACTIVE = 8

BB = 16


def _rim_kernel(x_ref, h_ref, wq_ref, wk_ref, wv_ref, wx_ref, wh_ref,
                bx_ref, bh_ref, out_ref):
    x = x_ref[...].reshape(BB * S, D_IN)
    h = h_ref[...].reshape(BB * NUM_K, HIDDEN)

    k = jnp.dot(x, wk_ref[...], preferred_element_type=jnp.float32)
    v = jnp.dot(x, wv_ref[...], preferred_element_type=jnp.float32)
    q = jnp.dot(h, wq_ref[...], preferred_element_type=jnp.float32)

    sim = jnp.dot(q, k.T, preferred_element_type=jnp.float32) * (
        1.0 / math.sqrt(D_K))
    row_b = jax.lax.broadcasted_iota(jnp.int32, sim.shape, 0) // NUM_K
    col_b = jax.lax.broadcasted_iota(jnp.int32, sim.shape, 1) // S
    sim = jnp.where(row_b == col_b, sim, -1e30)

    m = jnp.maximum(jnp.max(sim, axis=1, keepdims=True), 0.0)
    e = jnp.exp(sim - m)
    e_null = jnp.exp(-m)
    denom = jnp.sum(e, axis=1, keepdims=True) + e_null
    p = e / denom
    null_attn = (e_null / denom).reshape(BB, NUM_K)

    a = null_attn
    rank = jnp.zeros((BB, NUM_K), dtype=jnp.float32)
    col = jax.lax.broadcasted_iota(jnp.int32, (BB, NUM_K), 1)
    for j in range(NUM_K):
        aj = a[:, j:j + 1]
        cmp = (aj < a) | ((aj == a) & (j < col))
        rank = rank + cmp.astype(jnp.float32)
    mask = (rank < ACTIVE).astype(jnp.float32)
    mask_rows = mask.reshape(BB * NUM_K, 1)

    attended = jnp.dot(p, v, preferred_element_type=jnp.float32)
    attended = attended * mask_rows

    gates_x = jnp.dot(attended, wx_ref[...],
                      preferred_element_type=jnp.float32) + bx_ref[...]
    gates_h = jnp.dot(h, wh_ref[...],
                      preferred_element_type=jnp.float32) + bh_ref[...]
    xr = gates_x[:, :HIDDEN]
    xz = gates_x[:, HIDDEN:2 * HIDDEN]
    xn = gates_x[:, 2 * HIDDEN:]
    hr = gates_h[:, :HIDDEN]
    hz = gates_h[:, HIDDEN:2 * HIDDEN]
    hn = gates_h[:, 2 * HIDDEN:]
    r = jax.nn.sigmoid(xr + hr)
    z = jax.nn.sigmoid(xz + hz)
    n = jnp.tanh(xn + r * hn)
    new_h = (1.0 - z) * n + z * h
    out = mask_rows * new_h + (1.0 - mask_rows) * h
    out_ref[...] = out.reshape(BB, NUM_K, HIDDEN)


@functools.partial(jax.jit, static_argnames=("interpret",))
def _run(input, init_hidden, W_q, W_k, W_v, W_x, W_h, b_x, b_h,
         interpret=False):
    grid = (B // BB,)
    out = pl.pallas_call(
        _rim_kernel,
        grid=grid,
        in_specs=[
            pl.BlockSpec((BB, S, D_IN), lambda i: (i, 0, 0)),
            pl.BlockSpec((BB, NUM_K, HIDDEN), lambda i: (i, 0, 0)),
            pl.BlockSpec((HIDDEN, D_K), lambda i: (0, 0)),
            pl.BlockSpec((D_IN, D_K), lambda i: (0, 0)),
            pl.BlockSpec((D_IN, D_V), lambda i: (0, 0)),
            pl.BlockSpec((D_V, 3 * HIDDEN), lambda i: (0, 0)),
            pl.BlockSpec((HIDDEN, 3 * HIDDEN), lambda i: (0, 0)),
            pl.BlockSpec((1, 3 * HIDDEN), lambda i: (0, 0)),
            pl.BlockSpec((1, 3 * HIDDEN), lambda i: (0, 0)),
        ],
        out_specs=pl.BlockSpec((BB, NUM_K, HIDDEN), lambda i: (i, 0, 0)),
        out_shape=jax.ShapeDtypeStruct((B, NUM_K, HIDDEN), jnp.float32),
        interpret=interpret,
    )(input, init_hidden, W_q, W_k, W_v, W_x, W_h,
      b_x.reshape(1, 3 * HIDDEN), b_h.reshape(1, 3 * HIDDEN))
    return out


def kernel(input, init_hidden, W_q, W_k, W_v, W_x, W_h, b_x, b_h):
    return _run(input, init_hidden, W_q, W_k, W_v, W_x, W_h, b_x, b_h)
